# baseline (speedup 1.0000x reference)
import functools

import jax
import jax.numpy as jnp
from jax import lax
from jax.experimental import pallas as pl
from jax.experimental.pallas import tpu as pltpu
from jax.experimental.pallas import tpu_sc as plsc

_N = 1000
_B = 16384
_NC = 2
_NS = 16
_NW = _NC * _NS
_BPW = _B // _NW
_CT = 128
_NCT = _BPW // _CT
_H0 = 504
_H1 = _N - _H0
_L = 16

_mesh = plsc.VectorSubcoreMesh(core_axis_name="c", subcore_axis_name="s")


@functools.partial(
    pl.kernel,
    out_type=jax.ShapeDtypeStruct((_N, _B), jnp.float32),
    mesh=_mesh,
    compiler_params=pltpu.CompilerParams(needs_layout_passes=False),
    scratch_types=[
        pltpu.VMEM((_BPW,), jnp.int32),
        pltpu.VMEM((_H0, _CT), jnp.float32),
        pltpu.VMEM((_H0, _CT), jnp.float32),
        pltpu.SemaphoreType.DMA,
        pltpu.SemaphoreType.DMA,
    ],
)
def _onehot_t_sc(x_hbm, out_hbm, idx_v, buf0, buf1, sem0, sem1):
    wid = lax.axis_index("s") * _NC + lax.axis_index("c")
    base_col = wid * _BPW

    pltpu.sync_copy(x_hbm.at[pl.ds(base_col, _BPW)], idx_v)

    zeros = jnp.zeros((_L,), jnp.float32)
    ones = jnp.ones((_L,), jnp.float32)
    iota = lax.iota(jnp.int32, _L)

    def zero_fill(buf):
        def zero_body(i, carry):
            for r in range(8):
                for c in range(0, _CT, _L):
                    buf[i * 8 + r, pl.ds(c, _L)] = zeros
            return carry
        lax.fori_loop(0, _H0 // 8, zero_body, 0)

    def scatter(buf, ct, lo, hi, val):
        for k in range(_CT // _L):
            xv = idx_v[pl.ds(ct * _CT + k * _L, _L)]
            cols = iota + (k * _L)
            mask = (xv >= lo) & (xv < hi)
            plsc.store_scatter(buf, [xv - lo, cols], val, mask=mask)

    chunks = [(ct, h) for ct in range(_NCT) for h in range(2)]
    halves = ((0, _H0), (_H0, _N))
    bufs = (buf0, buf1)
    sems = (sem0, sem1)
    copies = [None, None]
    for i, (ct, h) in enumerate(chunks):
        b = i % 2
        lo, hi = halves[h]
        if i < 2:
            zero_fill(bufs[b])
        if copies[b] is not None:
            copies[b].wait()
            pct, ph = chunks[i - 2]
            plo, phi = halves[ph]
            scatter(bufs[b], pct, plo, phi, zeros)
        scatter(bufs[b], ct, lo, hi, ones)
        rows = hi - lo
        src = bufs[b] if rows == _H0 else bufs[b].at[pl.ds(0, rows)]
        dst = out_hbm.at[pl.ds(lo, rows), pl.ds(base_col + ct * _CT, _CT)]
        copies[b] = pltpu.async_copy(src, dst, sems[b])
    copies[0].wait()
    copies[1].wait()


def kernel(x):
    return _onehot_t_sc(x.astype(jnp.int32)).T


# speedup vs baseline: 1.0030x; 1.0030x over previous
import functools

import jax
import jax.numpy as jnp
from jax import lax
from jax.experimental import pallas as pl
from jax.experimental.pallas import tpu as pltpu
from jax.experimental.pallas import tpu_sc as plsc

_N = 1000
_B = 16384
_NC = 2
_NS = 16
_NW = _NC * _NS
_BPW = _B // _NW
_CT = 128
_NCT = _BPW // _CT
_H0 = 504
_H1 = _N - _H0
_L = 16

_mesh = plsc.VectorSubcoreMesh(core_axis_name="c", subcore_axis_name="s")


@functools.partial(
    pl.kernel,
    out_type=jax.ShapeDtypeStruct((_N, _B), jnp.float32),
    mesh=_mesh,
    compiler_params=pltpu.CompilerParams(needs_layout_passes=False),
    scratch_types=[
        pltpu.VMEM((_BPW,), jnp.int32),
        pltpu.VMEM((_H0, _CT), jnp.float32),
        pltpu.VMEM((_H0, _CT), jnp.float32),
        pltpu.SemaphoreType.DMA,
        pltpu.SemaphoreType.DMA,
    ],
)
def _onehot_t_sc(x_hbm, out_hbm, idx_v, buf0, buf1, sem0, sem1):
    wid = lax.axis_index("s") * _NC + lax.axis_index("c")
    base_col = wid * _BPW

    pltpu.sync_copy(x_hbm.at[pl.ds(base_col, _BPW)], idx_v)

    zeros = jnp.zeros((_L,), jnp.float32)
    ones = jnp.ones((_L,), jnp.float32)
    iota = lax.iota(jnp.int32, _L)

    def zero_fill(buf):
        def zero_body(r, carry):
            for c in range(0, _CT, _L):
                buf[r, pl.ds(c, _L)] = zeros
            return carry
        lax.fori_loop(0, _H0, zero_body, 0)

    def scatter(buf, ct, lo, hi, val):
        for k in range(_CT // _L):
            xv = idx_v[pl.ds(ct * _CT + k * _L, _L)]
            cols = iota + (k * _L)
            mask = (xv >= lo) & (xv < hi)
            plsc.store_scatter(buf, [xv - lo, cols], val, mask=mask)

    chunks = [(ct, h) for ct in range(_NCT) for h in range(2)]
    halves = ((0, _H0), (_H0, _N))
    bufs = (buf0, buf1)
    sems = (sem0, sem1)
    copies = [None, None]
    for i, (ct, h) in enumerate(chunks):
        b = i % 2
        lo, hi = halves[h]
        if i < 2:
            zero_fill(bufs[b])
        if copies[b] is not None:
            copies[b].wait()
            pct, ph = chunks[i - 2]
            plo, phi = halves[ph]
            scatter(bufs[b], pct, plo, phi, zeros)
        scatter(bufs[b], ct, lo, hi, ones)
        rows = hi - lo
        src = bufs[b] if rows == _H0 else bufs[b].at[pl.ds(0, rows)]
        dst = out_hbm.at[pl.ds(lo, rows), pl.ds(base_col + ct * _CT, _CT)]
        copies[b] = pltpu.async_copy(src, dst, sems[b])
    copies[0].wait()
    copies[1].wait()


def kernel(x):
    return _onehot_t_sc(x.astype(jnp.int32)).T
